# baseline (device time: 219354 ns/iter reference)
import jax
import jax.numpy as jnp
from jax import lax
from jax.experimental import pallas as pl
from jax.experimental.pallas import tpu as pltpu

N_DEV = 8


def kernel(A, B):
    m, k = A.shape
    k2, n = B.shape
    assert k == k2
    m_chunk = m // N_DEV
    n_hops = N_DEV - 1
    total_hops = 2 * n_hops

    def body(a_ref, b_ref, out_ref, comm_ref, send_sems, recv_sems):
        my = lax.axis_index("i")
        left = lax.rem(my + N_DEV - 1, N_DEV)
        right = lax.rem(my + 1, N_DEV)

        barrier_sem = pltpu.get_barrier_semaphore()
        for nbr in (left, right):
            pl.semaphore_signal(
                barrier_sem, inc=1,
                device_id=(nbr,), device_id_type=pl.DeviceIdType.MESH,
            )
        pl.semaphore_wait(barrier_sem, 2)

        out_ref[:, :] = jnp.dot(
            a_ref[:, :], b_ref[:, :], preferred_element_type=jnp.float32
        )

        def chunk_rows(c):
            return pl.ds(c * m_chunk, m_chunk)

        for s in range(n_hops):
            send_c = lax.rem(my - s + N_DEV, N_DEV)
            recv_c = lax.rem(my - s - 1 + N_DEV, N_DEV)
            rdma = pltpu.make_async_remote_copy(
                src_ref=out_ref.at[chunk_rows(send_c), :],
                dst_ref=comm_ref.at[s],
                send_sem=send_sems.at[s],
                recv_sem=recv_sems.at[s],
                device_id=(right,),
                device_id_type=pl.DeviceIdType.MESH,
            )
            rdma.start()
            rdma.wait()
            out_ref[chunk_rows(recv_c), :] += comm_ref[s, :, :]

        for t in range(n_hops):
            h = n_hops + t
            send_c = lax.rem(my + 1 - t + N_DEV, N_DEV)
            recv_c = lax.rem(my - t + N_DEV, N_DEV)
            rdma = pltpu.make_async_remote_copy(
                src_ref=out_ref.at[chunk_rows(send_c), :],
                dst_ref=comm_ref.at[h],
                send_sem=send_sems.at[h],
                recv_sem=recv_sems.at[h],
                device_id=(right,),
                device_id_type=pl.DeviceIdType.MESH,
            )
            rdma.start()
            rdma.wait()
            out_ref[chunk_rows(recv_c), :] = comm_ref[h, :, :]

    return pl.pallas_call(
        body,
        out_shape=jax.ShapeDtypeStruct((m, n), jnp.float32),
        in_specs=[
            pl.BlockSpec(memory_space=pltpu.VMEM),
            pl.BlockSpec(memory_space=pltpu.VMEM),
        ],
        out_specs=pl.BlockSpec(memory_space=pltpu.VMEM),
        scratch_shapes=[
            pltpu.VMEM((total_hops, m_chunk, n), jnp.float32),
            pltpu.SemaphoreType.DMA((total_hops,)),
            pltpu.SemaphoreType.DMA((total_hops,)),
        ],
        compiler_params=pltpu.CompilerParams(collective_id=0),
    )(A, B)


# device time: 86175 ns/iter; 2.5454x vs baseline; 2.5454x over previous
import jax
import jax.numpy as jnp
from jax import lax
from jax.experimental import pallas as pl
from jax.experimental.pallas import tpu as pltpu

N_DEV = 8
N_GROUPS = 3
ORDERS = ((0, 1, 2), (1, 2, 0), (2, 0, 1))


def kernel(A, B):
    m, k = A.shape
    k2, n = B.shape
    assert k == k2
    g_rows = m // N_GROUPS

    def body(a_ref, b_ref, out_ref,
             rs0, rs1, rs2, ag0, ag1, ag2,
             rs_send, rs_recv, ag_send, ag_recv):
        p = lax.axis_index("i")
        plane = lax.rem(p, 4)
        zc = lax.div(p, 4)
        bx = jnp.where((plane == 1) | (plane == 2), 1, 0)
        by = jnp.where(plane >= 2, 1, 0)
        bz = zc
        nx = jnp.bitwise_xor(p, 1)
        ny = 4 * zc + (3 - plane)
        nz = jnp.bitwise_xor(p, 4)
        ax = ((nx, bx), (ny, by), (nz, bz))

        barrier_sem = pltpu.get_barrier_semaphore()
        for nbr, _ in ax:
            pl.semaphore_signal(
                barrier_sem, inc=1,
                device_id=(nbr,), device_id_type=pl.DeviceIdType.MESH,
            )
        pl.semaphore_wait(barrier_sem, 3)

        out_ref[:, :] = jnp.dot(
            a_ref[:, :], b_ref[:, :], preferred_element_type=jnp.float32
        )

        rs_comm = (rs0, rs1, rs2)
        ag_comm = (ag0, ag1, ag2)

        rb = [512 * g for g in range(N_GROUPS)]
        for s in range(3):
            blk = 256 >> s
            rdmas = []
            for g in range(N_GROUPS):
                nbr, bit = ax[ORDERS[g][s]]
                send_base = rb[g] + (1 - bit) * blk
                rdma = pltpu.make_async_remote_copy(
                    src_ref=out_ref.at[pl.ds(send_base, blk), :],
                    dst_ref=rs_comm[s].at[g],
                    send_sem=rs_send.at[s, g],
                    recv_sem=rs_recv.at[s, g],
                    device_id=(nbr,),
                    device_id_type=pl.DeviceIdType.MESH,
                )
                rdma.start()
                rdmas.append(rdma)
                rb[g] = rb[g] + bit * blk
            for g in range(N_GROUPS):
                rdmas[g].wait()
            for g in range(N_GROUPS):
                out_ref[pl.ds(rb[g], blk), :] += rs_comm[s][g, :, :]


        vb = rb
        for j in range(3):
            blk = 64 << j
            s = 2 - j
            rdmas = []
            pvb = []
            for g in range(N_GROUPS):
                nbr, bit = ax[ORDERS[g][s]]
                pvb.append(vb[g] + (1 - 2 * bit) * blk)
                rdma = pltpu.make_async_remote_copy(
                    src_ref=out_ref.at[pl.ds(vb[g], blk), :],
                    dst_ref=ag_comm[j].at[g],
                    send_sem=ag_send.at[j, g],
                    recv_sem=ag_recv.at[j, g],
                    device_id=(nbr,),
                    device_id_type=pl.DeviceIdType.MESH,
                )
                rdma.start()
                rdmas.append(rdma)
            for g in range(N_GROUPS):
                rdmas[g].wait()
            for g in range(N_GROUPS):
                out_ref[pl.ds(pvb[g], blk), :] = ag_comm[j][g, :, :]
                _, bit = ax[ORDERS[g][s]]
                vb[g] = vb[g] - bit * blk

    return pl.pallas_call(
        body,
        out_shape=jax.ShapeDtypeStruct((m, n), jnp.float32),
        in_specs=[
            pl.BlockSpec(memory_space=pltpu.VMEM),
            pl.BlockSpec(memory_space=pltpu.VMEM),
        ],
        out_specs=pl.BlockSpec(memory_space=pltpu.VMEM),
        scratch_shapes=[
            pltpu.VMEM((N_GROUPS, 256, n), jnp.float32),
            pltpu.VMEM((N_GROUPS, 128, n), jnp.float32),
            pltpu.VMEM((N_GROUPS, 64, n), jnp.float32),
            pltpu.VMEM((N_GROUPS, 64, n), jnp.float32),
            pltpu.VMEM((N_GROUPS, 128, n), jnp.float32),
            pltpu.VMEM((N_GROUPS, 256, n), jnp.float32),
            pltpu.SemaphoreType.DMA((3, N_GROUPS)),
            pltpu.SemaphoreType.DMA((3, N_GROUPS)),
            pltpu.SemaphoreType.DMA((3, N_GROUPS)),
            pltpu.SemaphoreType.DMA((3, N_GROUPS)),
        ],
        compiler_params=pltpu.CompilerParams(collective_id=0),
    )(A, B)


# device time: 83980 ns/iter; 2.6120x vs baseline; 1.0261x over previous
import jax
import jax.numpy as jnp
from jax import lax
from jax.experimental import pallas as pl
from jax.experimental.pallas import tpu as pltpu

N_DEV = 8
N_GROUPS = 3
ORDERS = ((0, 1, 2), (1, 2, 0), (2, 0, 1))


def kernel(A, B):
    m, k = A.shape
    k2, n = B.shape
    assert k == k2
    g_rows = m // N_GROUPS
    h_rows = g_rows // 2

    def body(a_ref, b_ref, out_ref,
             rs0, rs1, rs2,
             rs_send, rs_recv, ag_send, ag_recv):
        p = lax.axis_index("i")
        plane = lax.rem(p, 4)
        zc = lax.div(p, 4)
        bx = jnp.where((plane == 1) | (plane == 2), 1, 0)
        by = jnp.where(plane >= 2, 1, 0)
        bz = zc
        nx = jnp.bitwise_xor(p, 1)
        ny = 4 * zc + (3 - plane)
        nz = jnp.bitwise_xor(p, 4)
        ax = ((nx, bx), (ny, by), (nz, bz))

        barrier_sem = pltpu.get_barrier_semaphore()
        for nbr, _ in ax:
            pl.semaphore_signal(
                barrier_sem, inc=1,
                device_id=(nbr,), device_id_type=pl.DeviceIdType.MESH,
            )
        pl.semaphore_wait(barrier_sem, 3)

        rs_comm = (rs0, rs1, rs2)

        def matmul_rows(base, rows):
            out_ref[pl.ds(base, rows), :] = jnp.dot(
                a_ref[pl.ds(base, rows), :], b_ref[:, :],
                preferred_element_type=jnp.float32,
            )

        def start_rs(g, s, rb_g):
            blk = h_rows >> s
            nbr, bit = ax[ORDERS[g][s]]
            rdma = pltpu.make_async_remote_copy(
                src_ref=out_ref.at[pl.ds(rb_g + (1 - bit) * blk, blk), :],
                dst_ref=rs_comm[s].at[g],
                send_sem=rs_send.at[s, g],
                recv_sem=rs_recv.at[s, g],
                device_id=(nbr,),
                device_id_type=pl.DeviceIdType.MESH,
            )
            rdma.start()
            return rdma, rb_g + bit * blk

        def start_ag(g, j, vb_g):
            blk = (g_rows // N_DEV) << j
            nbr, bit = ax[ORDERS[g][2 - j]]
            rdma = pltpu.make_async_remote_copy(
                src_ref=out_ref.at[pl.ds(vb_g, blk), :],
                dst_ref=out_ref.at[pl.ds(vb_g, blk), :],
                send_sem=ag_send.at[j, g],
                recv_sem=ag_recv.at[j, g],
                device_id=(nbr,),
                device_id_type=pl.DeviceIdType.MESH,
            )
            rdma.start()
            return rdma, bit

        rb = [512 * g for g in range(N_GROUPS)]
        rdmas = [None] * N_GROUPS
        for g in range(N_GROUPS):
            _, bit = ax[ORDERS[g][0]]
            matmul_rows(rb[g] + (1 - bit) * h_rows, h_rows)
            rdmas[g], rb[g] = start_rs(g, 0, rb[g])
        for g in range(N_GROUPS):
            matmul_rows(rb[g], h_rows)

        for s in range(3):
            blk = h_rows >> s
            for g in range(N_GROUPS):
                rdmas[g].wait()
                out_ref[pl.ds(rb[g], blk), :] += rs_comm[s][g, :, :]
                if s < 2:
                    rdmas[g], rb[g] = start_rs(g, s + 1, rb[g])
                else:
                    rdmas[g], _ = start_ag(g, 0, rb[g])

        vb = rb
        for j in range(3):
            blk = (g_rows // N_DEV) << j
            for g in range(N_GROUPS):
                rdmas[g].wait()
                _, bit = ax[ORDERS[g][2 - j]]
                vb[g] = vb[g] - bit * blk
                if j < 2:
                    rdmas[g], _ = start_ag(g, j + 1, vb[g])

    return pl.pallas_call(
        body,
        out_shape=jax.ShapeDtypeStruct((m, n), jnp.float32),
        in_specs=[
            pl.BlockSpec(memory_space=pltpu.VMEM),
            pl.BlockSpec(memory_space=pltpu.VMEM),
        ],
        out_specs=pl.BlockSpec(memory_space=pltpu.VMEM),
        scratch_shapes=[
            pltpu.VMEM((N_GROUPS, 256, n), jnp.float32),
            pltpu.VMEM((N_GROUPS, 128, n), jnp.float32),
            pltpu.VMEM((N_GROUPS, 64, n), jnp.float32),
            pltpu.SemaphoreType.DMA((3, N_GROUPS)),
            pltpu.SemaphoreType.DMA((3, N_GROUPS)),
            pltpu.SemaphoreType.DMA((3, N_GROUPS)),
            pltpu.SemaphoreType.DMA((3, N_GROUPS)),
        ],
        compiler_params=pltpu.CompilerParams(collective_id=0),
    )(A, B)


# device time: 54508 ns/iter; 4.0243x vs baseline; 1.5407x over previous
import jax
import jax.numpy as jnp
from jax import lax
from jax.experimental import pallas as pl
from jax.experimental.pallas import tpu as pltpu

N_DEV = 8
N_GROUPS = 3
ORDERS = ((0, 1, 2), (1, 2, 0), (2, 0, 1))


def kernel(A, B):
    m, k = A.shape
    k2, n = B.shape
    assert k == k2
    g_rows = m // N_GROUPS
    h_rows = g_rows // 2
    c_rows = g_rows // N_DEV

    def body(a_ref, b_ref, out_ref, mir_ref,
             rs0, rs1, rs2,
             rs_send, rs_recv, ag_send, ag_recv):
        p = lax.axis_index("i")
        plane = lax.rem(p, 4)
        zc = lax.div(p, 4)
        bx = jnp.where((plane == 1) | (plane == 2), 1, 0)
        by = jnp.where(plane >= 2, 1, 0)
        bz = zc
        nx = jnp.bitwise_xor(p, 1)
        ny = 4 * zc + (3 - plane)
        nz = jnp.bitwise_xor(p, 4)
        ax = ((nx, bx), (ny, by), (nz, bz))

        barrier_sem = pltpu.get_barrier_semaphore()
        for nbr, _ in ax:
            pl.semaphore_signal(
                barrier_sem, inc=1,
                device_id=(nbr,), device_id_type=pl.DeviceIdType.MESH,
            )
        pl.semaphore_wait(barrier_sem, 3)

        rs_comm = (rs0, rs1, rs2)

        def matmul_rows(base, rows):
            out_ref[pl.ds(base, rows), :] = jnp.dot(
                a_ref[pl.ds(base, rows), :], b_ref[:, :],
                preferred_element_type=jnp.float32,
            )

        def start_rs(g, s, rb_g):
            blk = h_rows >> s
            nbr, bit = ax[ORDERS[g][s]]
            sb = rb_g + (1 - bit) * blk
            mir_ref[pl.ds(sb, blk), :] = out_ref[pl.ds(sb, blk), :].astype(
                jnp.bfloat16
            )
            rdma = pltpu.make_async_remote_copy(
                src_ref=mir_ref.at[pl.ds(sb, blk), :],
                dst_ref=rs_comm[s].at[g],
                send_sem=rs_send.at[s, g],
                recv_sem=rs_recv.at[s, g],
                device_id=(nbr,),
                device_id_type=pl.DeviceIdType.MESH,
            )
            rdma.start()
            return rdma, rb_g + bit * blk

        def start_ag(g, j, vb_g):
            blk = c_rows << j
            nbr, bit = ax[ORDERS[g][2 - j]]
            rdma = pltpu.make_async_remote_copy(
                src_ref=mir_ref.at[pl.ds(vb_g, blk), :],
                dst_ref=mir_ref.at[pl.ds(vb_g, blk), :],
                send_sem=ag_send.at[j, g],
                recv_sem=ag_recv.at[j, g],
                device_id=(nbr,),
                device_id_type=pl.DeviceIdType.MESH,
            )
            rdma.start()
            return rdma, bit

        rb = [512 * g for g in range(N_GROUPS)]
        rdmas = [None] * N_GROUPS
        for g in range(N_GROUPS):
            _, bit = ax[ORDERS[g][0]]
            matmul_rows(rb[g] + (1 - bit) * h_rows, h_rows)
            rdmas[g], rb[g] = start_rs(g, 0, rb[g])
        for g in range(N_GROUPS):
            matmul_rows(rb[g], h_rows)

        for s in range(3):
            blk = h_rows >> s
            for g in range(N_GROUPS):
                rdmas[g].wait()
                out_ref[pl.ds(rb[g], blk), :] += rs_comm[s][
                    g, :, :
                ].astype(jnp.float32)
                if s < 2:
                    rdmas[g], rb[g] = start_rs(g, s + 1, rb[g])
                else:
                    mir_ref[pl.ds(rb[g], c_rows), :] = out_ref[
                        pl.ds(rb[g], c_rows), :
                    ].astype(jnp.bfloat16)
                    rdmas[g], _ = start_ag(g, 0, rb[g])

        vb = rb
        for j in range(3):
            blk = c_rows << j
            for g in range(N_GROUPS):
                rdmas[g].wait()
                _, bit = ax[ORDERS[g][2 - j]]
                pvb = vb[g] + (1 - 2 * bit) * blk
                vb[g] = vb[g] - bit * blk
                if j < 2:
                    rdmas[g], _ = start_ag(g, j + 1, vb[g])
                out_ref[pl.ds(pvb, blk), :] = mir_ref[
                    pl.ds(pvb, blk), :
                ].astype(jnp.float32)

    return pl.pallas_call(
        body,
        out_shape=jax.ShapeDtypeStruct((m, n), jnp.float32),
        in_specs=[
            pl.BlockSpec(memory_space=pltpu.VMEM),
            pl.BlockSpec(memory_space=pltpu.VMEM),
        ],
        out_specs=pl.BlockSpec(memory_space=pltpu.VMEM),
        scratch_shapes=[
            pltpu.VMEM((m, n), jnp.bfloat16),
            pltpu.VMEM((N_GROUPS, 256, n), jnp.bfloat16),
            pltpu.VMEM((N_GROUPS, 128, n), jnp.bfloat16),
            pltpu.VMEM((N_GROUPS, 64, n), jnp.bfloat16),
            pltpu.SemaphoreType.DMA((3, N_GROUPS)),
            pltpu.SemaphoreType.DMA((3, N_GROUPS)),
            pltpu.SemaphoreType.DMA((3, N_GROUPS)),
            pltpu.SemaphoreType.DMA((3, N_GROUPS)),
        ],
        compiler_params=pltpu.CompilerParams(collective_id=0),
    )(A, B)


# device time: 46683 ns/iter; 4.6988x vs baseline; 1.1676x over previous
import jax
import jax.numpy as jnp
from jax import lax
from jax.experimental import pallas as pl
from jax.experimental.pallas import tpu as pltpu

N_DEV = 8
N_GROUPS = 6
ORDERS = ((0, 1, 2), (1, 2, 0), (2, 0, 1), (0, 1, 2), (1, 2, 0), (2, 0, 1))


def kernel(A, B):
    m, k = A.shape
    k2, n = B.shape
    assert k == k2
    g_rows = m // N_GROUPS
    h_rows = g_rows // 2
    c_rows = g_rows // N_DEV

    def body(a_ref, b_ref, out_ref, mir_ref,
             rs0, rs1, rs2,
             rs_send, rs_recv, ag_send, ag_recv):
        p = lax.axis_index("i")
        plane = lax.rem(p, 4)
        zc = lax.div(p, 4)
        bx = jnp.where((plane == 1) | (plane == 2), 1, 0)
        by = jnp.where(plane >= 2, 1, 0)
        bz = zc
        nx = jnp.bitwise_xor(p, 1)
        ny = 4 * zc + (3 - plane)
        nz = jnp.bitwise_xor(p, 4)
        ax = ((nx, bx), (ny, by), (nz, bz))

        barrier_sem = pltpu.get_barrier_semaphore()
        for nbr, _ in ax:
            pl.semaphore_signal(
                barrier_sem, inc=1,
                device_id=(nbr,), device_id_type=pl.DeviceIdType.MESH,
            )
        pl.semaphore_wait(barrier_sem, 3)

        rs_comm = (rs0, rs1, rs2)

        def matmul_rows(base, rows):
            out_ref[pl.ds(base, rows), :] = jnp.dot(
                a_ref[pl.ds(base, rows), :], b_ref[:, :],
                preferred_element_type=jnp.float32,
            )

        def start_rs(g, s, rb_g):
            blk = h_rows >> s
            nbr, bit = ax[ORDERS[g][s]]
            sb = rb_g + (1 - bit) * blk
            mir_ref[pl.ds(sb, blk), :] = out_ref[pl.ds(sb, blk), :].astype(
                jnp.bfloat16
            )
            rdma = pltpu.make_async_remote_copy(
                src_ref=mir_ref.at[pl.ds(sb, blk), :],
                dst_ref=rs_comm[s].at[g],
                send_sem=rs_send.at[s, g],
                recv_sem=rs_recv.at[s, g],
                device_id=(nbr,),
                device_id_type=pl.DeviceIdType.MESH,
            )
            rdma.start()
            return rdma, rb_g + bit * blk

        def start_ag(g, j, vb_g):
            blk = c_rows << j
            nbr, bit = ax[ORDERS[g][2 - j]]
            rdma = pltpu.make_async_remote_copy(
                src_ref=mir_ref.at[pl.ds(vb_g, blk), :],
                dst_ref=mir_ref.at[pl.ds(vb_g, blk), :],
                send_sem=ag_send.at[j, g],
                recv_sem=ag_recv.at[j, g],
                device_id=(nbr,),
                device_id_type=pl.DeviceIdType.MESH,
            )
            rdma.start()
            return rdma, bit

        rb = [g_rows * g for g in range(N_GROUPS)]
        rdmas = [None] * N_GROUPS
        for g in range(N_GROUPS):
            _, bit = ax[ORDERS[g][0]]
            matmul_rows(rb[g] + (1 - bit) * h_rows, h_rows)
            rdmas[g], rb[g] = start_rs(g, 0, rb[g])
        for g in range(N_GROUPS):
            matmul_rows(rb[g], h_rows)

        for s in range(3):
            blk = h_rows >> s
            for g in range(N_GROUPS):
                rdmas[g].wait()
                out_ref[pl.ds(rb[g], blk), :] += rs_comm[s][
                    g, :, :
                ].astype(jnp.float32)
                if s < 2:
                    rdmas[g], rb[g] = start_rs(g, s + 1, rb[g])
                else:
                    mir_ref[pl.ds(rb[g], c_rows), :] = out_ref[
                        pl.ds(rb[g], c_rows), :
                    ].astype(jnp.bfloat16)
                    rdmas[g], _ = start_ag(g, 0, rb[g])

        vb = rb
        for j in range(3):
            blk = c_rows << j
            for g in range(N_GROUPS):
                rdmas[g].wait()
                _, bit = ax[ORDERS[g][2 - j]]
                pvb = vb[g] + (1 - 2 * bit) * blk
                vb[g] = vb[g] - bit * blk
                if j < 2:
                    rdmas[g], _ = start_ag(g, j + 1, vb[g])
                out_ref[pl.ds(pvb, blk), :] = mir_ref[
                    pl.ds(pvb, blk), :
                ].astype(jnp.float32)

    return pl.pallas_call(
        body,
        out_shape=jax.ShapeDtypeStruct((m, n), jnp.float32),
        in_specs=[
            pl.BlockSpec(memory_space=pltpu.VMEM),
            pl.BlockSpec(memory_space=pltpu.VMEM),
        ],
        out_specs=pl.BlockSpec(memory_space=pltpu.VMEM),
        scratch_shapes=[
            pltpu.VMEM((m, n), jnp.bfloat16),
            pltpu.VMEM((N_GROUPS, h_rows, n), jnp.bfloat16),
            pltpu.VMEM((N_GROUPS, h_rows // 2, n), jnp.bfloat16),
            pltpu.VMEM((N_GROUPS, h_rows // 4, n), jnp.bfloat16),
            pltpu.SemaphoreType.DMA((3, N_GROUPS)),
            pltpu.SemaphoreType.DMA((3, N_GROUPS)),
            pltpu.SemaphoreType.DMA((3, N_GROUPS)),
            pltpu.SemaphoreType.DMA((3, N_GROUPS)),
        ],
        compiler_params=pltpu.CompilerParams(collective_id=0),
    )(A, B)
